# baseline (speedup 1.0000x reference)
import functools

import jax
import jax.numpy as jnp
from jax.experimental import pallas as pl
from jax.experimental.pallas import tpu as pltpu

_MIB = 1024 * 1024


def _se_block_kernel(x_ref, w1_ref, b1_ref, w2_ref, b2_ref, o_ref, *, inv_hw):
    xf = x_ref[...]
    avg = jnp.sum(xf, axis=-1, keepdims=True) * inv_hw
    h = jnp.dot(w1_ref[...], avg,
                preferred_element_type=jnp.float32) + b1_ref[...]
    h = jnp.where(h >= 0.0, h, 0.2 * h)
    s = jnp.dot(w2_ref[...], h,
                preferred_element_type=jnp.float32) + b2_ref[...]
    g = jax.nn.sigmoid(s)
    o_ref[...] = (xf * g).astype(o_ref.dtype)


def _pick_batch_block(n, c, hw, itemsize, budget_bytes):
    best = 1
    for b in range(1, n + 1):
        if n % b:
            continue
        if b * c * hw * itemsize <= budget_bytes:
            best = b
    return best


@jax.jit
def _se_forward(x_nchw, w1, b1, w2, b2):
    N, C, H, W = x_nchw.shape
    Cr = w1.shape[0]
    HW = H * W
    x2 = x_nchw.reshape(N * C, HW)

    B = _pick_batch_block(N, C, HW, x_nchw.dtype.itemsize, 8 * _MIB)

    eye = jnp.eye(B, dtype=jnp.float32)
    w1bd = jnp.kron(eye, w1.astype(jnp.float32))
    w2bd = jnp.kron(eye, w2.astype(jnp.float32))
    b1bd = jnp.tile(b1.astype(jnp.float32), B).reshape(B * Cr, 1)
    b2bd = jnp.tile(b2.astype(jnp.float32), B).reshape(B * C, 1)

    BC = B * C
    BCr = B * Cr
    out = pl.pallas_call(
        functools.partial(_se_block_kernel, inv_hw=1.0 / HW),
        out_shape=jax.ShapeDtypeStruct((N * C, HW), x_nchw.dtype),
        grid=(N // B,),
        in_specs=[
            pl.BlockSpec((BC, HW), lambda i: (i, 0)),
            pl.BlockSpec((BCr, BC), lambda i: (0, 0)),
            pl.BlockSpec((BCr, 1), lambda i: (0, 0)),
            pl.BlockSpec((BC, BCr), lambda i: (0, 0)),
            pl.BlockSpec((BC, 1), lambda i: (0, 0)),
        ],
        out_specs=pl.BlockSpec((BC, HW), lambda i: (i, 0)),
        compiler_params=pltpu.CompilerParams(
            dimension_semantics=("parallel",),
            vmem_limit_bytes=56 * _MIB),
    )(x2, w1bd, b1bd, w2bd, b2bd)
    return out.reshape(N, C, H, W)


def kernel(x_nchw, w1, b1, w2, b2):
    return _se_forward(x_nchw, w1, b1, w2, b2)


# speedup vs baseline: 1.6508x; 1.6508x over previous
import functools

import jax
import jax.numpy as jnp
from jax.experimental import pallas as pl
from jax.experimental.pallas import tpu as pltpu

_MIB = 1024 * 1024


def _se_block_kernel(x_ref, w1_ref, b1_ref, w2_ref, b2_ref, o_ref, *, inv_hw):
    b, c, hw = x_ref.shape
    xf = x_ref[...].reshape(b * c, hw)
    avg = jnp.sum(xf, axis=-1, keepdims=True) * inv_hw
    h = jnp.dot(w1_ref[...], avg,
                preferred_element_type=jnp.float32) + b1_ref[...]
    h = jnp.where(h >= 0.0, h, 0.2 * h)
    s = jnp.dot(w2_ref[...], h,
                preferred_element_type=jnp.float32) + b2_ref[...]
    g = jax.nn.sigmoid(s)
    o_ref[...] = (xf * g).astype(o_ref.dtype).reshape(b, c, hw)


def _pick_batch_block(n, c, hw, itemsize, budget_bytes):
    best = 1
    for b in range(1, n + 1):
        if n % b:
            continue
        if b * c * hw * itemsize <= budget_bytes:
            best = b
    return best


@jax.jit
def _se_forward(x_nchw, w1, b1, w2, b2):
    N, C, H, W = x_nchw.shape
    Cr = w1.shape[0]
    HW = H * W
    x3 = x_nchw.reshape(N, C, HW)

    B = _pick_batch_block(N, C, HW, x_nchw.dtype.itemsize, 8 * _MIB)

    eye = jnp.eye(B, dtype=jnp.float32)
    w1bd = jnp.kron(eye, w1.astype(jnp.float32))
    w2bd = jnp.kron(eye, w2.astype(jnp.float32))
    b1bd = jnp.tile(b1.astype(jnp.float32), B).reshape(B * Cr, 1)
    b2bd = jnp.tile(b2.astype(jnp.float32), B).reshape(B * C, 1)

    BC = B * C
    BCr = B * Cr
    out = pl.pallas_call(
        functools.partial(_se_block_kernel, inv_hw=1.0 / HW),
        out_shape=jax.ShapeDtypeStruct((N, C, HW), x_nchw.dtype),
        grid=(N // B,),
        in_specs=[
            pl.BlockSpec((B, C, HW), lambda i: (i, 0, 0)),
            pl.BlockSpec((BCr, BC), lambda i: (0, 0)),
            pl.BlockSpec((BCr, 1), lambda i: (0, 0)),
            pl.BlockSpec((BC, BCr), lambda i: (0, 0)),
            pl.BlockSpec((BC, 1), lambda i: (0, 0)),
        ],
        out_specs=pl.BlockSpec((B, C, HW), lambda i: (i, 0, 0)),
        compiler_params=pltpu.CompilerParams(
            dimension_semantics=("parallel",),
            vmem_limit_bytes=56 * _MIB),
    )(x3, w1bd, b1bd, w2bd, b2bd)
    return out.reshape(N, C, H, W)


def kernel(x_nchw, w1, b1, w2, b2):
    return _se_forward(x_nchw, w1, b1, w2, b2)


# speedup vs baseline: 1.7117x; 1.0369x over previous
import functools

import jax
import jax.numpy as jnp
from jax.experimental import pallas as pl
from jax.experimental.pallas import tpu as pltpu

_MIB = 1024 * 1024


def _se_block_kernel(x_ref, w1_ref, b1_ref, w2_ref, b2_ref, o_ref, *, inv_hw):
    b, c, hw = x_ref.shape
    xf = x_ref[...]
    x2 = xf.reshape(b * c, hw)
    sums = jnp.sum(x2, axis=-1, keepdims=True)
    avg = (sums * inv_hw).reshape(b, c, 1)

    w1 = w1_ref[...]
    w2 = w2_ref[...]
    b1c = b1_ref[...].T
    b2c = b2_ref[...].T

    gates = []
    for i in range(b):
        h = jnp.dot(w1, avg[i], preferred_element_type=jnp.float32) + b1c
        h = jnp.where(h >= 0.0, h, 0.2 * h)
        s = jnp.dot(w2, h, preferred_element_type=jnp.float32) + b2c
        gates.append(jax.nn.sigmoid(s))
    g = jnp.stack(gates, axis=0)
    o_ref[...] = (xf * g).astype(o_ref.dtype)


def _pick_batch_block(n, c, hw, itemsize, budget_bytes):
    best = 1
    for b in range(1, n + 1):
        if n % b:
            continue
        if b * c * hw * itemsize <= budget_bytes:
            best = b
    return best


@jax.jit
def _se_forward(x_nchw, w1, b1, w2, b2):
    N, C, H, W = x_nchw.shape
    Cr = w1.shape[0]
    HW = H * W
    x3 = x_nchw.reshape(N, C, HW)
    b1r = b1.reshape(1, Cr)
    b2r = b2.reshape(1, C)

    B = _pick_batch_block(N, C, HW, x_nchw.dtype.itemsize, 8 * _MIB)

    out = pl.pallas_call(
        functools.partial(_se_block_kernel, inv_hw=1.0 / HW),
        out_shape=jax.ShapeDtypeStruct((N, C, HW), x_nchw.dtype),
        grid=(N // B,),
        in_specs=[
            pl.BlockSpec((B, C, HW), lambda i: (i, 0, 0)),
            pl.BlockSpec((Cr, C), lambda i: (0, 0)),
            pl.BlockSpec((1, Cr), lambda i: (0, 0)),
            pl.BlockSpec((C, Cr), lambda i: (0, 0)),
            pl.BlockSpec((1, C), lambda i: (0, 0)),
        ],
        out_specs=pl.BlockSpec((B, C, HW), lambda i: (i, 0, 0)),
        compiler_params=pltpu.CompilerParams(
            dimension_semantics=("parallel",),
            vmem_limit_bytes=56 * _MIB),
    )(x3, w1, b1r, w2, b2r)
    return out.reshape(N, C, H, W)


def kernel(x_nchw, w1, b1, w2, b2):
    return _se_forward(x_nchw, w1, b1, w2, b2)


# speedup vs baseline: 5.0170x; 2.9310x over previous
import functools

import jax
import jax.numpy as jnp
from jax import lax
from jax.experimental import pallas as pl
from jax.experimental.pallas import tpu as pltpu

_MIB = 1024 * 1024


def _pool_kernel(x_ref, o_ref, acc_ref):
    t = pl.program_id(1)

    @pl.when(t == 0)
    def _():
        acc_ref[...] = jnp.zeros_like(acc_ref)

    acc_ref[...] += jnp.sum(x_ref[...], axis=0)

    @pl.when(t == pl.num_programs(1) - 1)
    def _():
        o_ref[0] = acc_ref[...]


def _scale_kernel(x_ref, p_ref, w1_ref, b1_ref, w2t_ref, b2_ref, o_ref,
                  g_ref, *, inv_hw):
    t = pl.program_id(1)

    @pl.when(t == 0)
    def _():
        avg = (p_ref[0] + p_ref[1]) * inv_hw
        h = lax.dot_general(avg, w1_ref[...], (((1,), (1,)), ((), ())),
                            preferred_element_type=jnp.float32) + b1_ref[...]
        h = jnp.where(h >= 0.0, h, 0.2 * h)
        s = jnp.dot(h, w2t_ref[...],
                    preferred_element_type=jnp.float32) + b2_ref[...]
        g_ref[...] = jax.nn.sigmoid(s)

    o_ref[...] = x_ref[...] * g_ref[...]


def _pick_hw_tile(hw_half, n, c, itemsize, budget_bytes):
    best = 1
    for t in range(1, hw_half + 1):
        if hw_half % t:
            continue
        if t * n * c * itemsize <= budget_bytes:
            best = t
    return best


@jax.jit
def _se_forward(x_nchw, w1, b1, w2, b2):
    N, C, H, W = x_nchw.shape
    Cr = w1.shape[0]
    HW = H * W

    xs = jnp.transpose(x_nchw, (2, 3, 0, 1)).reshape(HW, N, C)
    w2t = w2.T
    b1r = b1.reshape(1, Cr)
    b2r = b2.reshape(1, C)

    half = HW // 2
    T = _pick_hw_tile(half, N, C, x_nchw.dtype.itemsize, 10 * _MIB)
    nT = half // T

    partial = pl.pallas_call(
        _pool_kernel,
        out_shape=jax.ShapeDtypeStruct((2, N, C), jnp.float32),
        grid=(2, nT),
        in_specs=[pl.BlockSpec((T, N, C), lambda i, t: (i * nT + t, 0, 0))],
        out_specs=pl.BlockSpec((1, N, C), lambda i, t: (i, 0, 0)),
        scratch_shapes=[pltpu.VMEM((N, C), jnp.float32)],
        compiler_params=pltpu.CompilerParams(
            dimension_semantics=("parallel", "arbitrary"),
            vmem_limit_bytes=56 * _MIB),
    )(xs)

    out = pl.pallas_call(
        functools.partial(_scale_kernel, inv_hw=1.0 / HW),
        out_shape=jax.ShapeDtypeStruct((HW, N, C), x_nchw.dtype),
        grid=(2, nT),
        in_specs=[
            pl.BlockSpec((T, N, C), lambda i, t: (i * nT + t, 0, 0)),
            pl.BlockSpec((2, N, C), lambda i, t: (0, 0, 0)),
            pl.BlockSpec((Cr, C), lambda i, t: (0, 0)),
            pl.BlockSpec((1, Cr), lambda i, t: (0, 0)),
            pl.BlockSpec((Cr, C), lambda i, t: (0, 0)),
            pl.BlockSpec((1, C), lambda i, t: (0, 0)),
        ],
        out_specs=pl.BlockSpec((T, N, C), lambda i, t: (i * nT + t, 0, 0)),
        scratch_shapes=[pltpu.VMEM((N, C), jnp.float32)],
        compiler_params=pltpu.CompilerParams(
            dimension_semantics=("parallel", "arbitrary"),
            vmem_limit_bytes=56 * _MIB),
    )(xs, partial, w1, b1r, w2t, b2r)

    return out.reshape(H, W, N, C).transpose(2, 3, 0, 1)


def kernel(x_nchw, w1, b1, w2, b2):
    return _se_forward(x_nchw, w1, b1, w2, b2)


# speedup vs baseline: 5.8816x; 1.1723x over previous
import functools

import jax
import jax.numpy as jnp
from jax import lax
from jax.experimental import pallas as pl
from jax.experimental.pallas import tpu as pltpu

_MIB = 1024 * 1024


def _se_kernel(x_ref, w1_ref, b1_ref, w2t_ref, b2_ref, o_ref,
               cache_ref, acc_ref, g_ref, *, n_t, tile, inv_hw):
    t = pl.program_id(1)

    @pl.when(t == 0)
    def _():
        acc_ref[...] = jnp.zeros_like(acc_ref)

    @pl.when(t < n_t)
    def _():
        xv = x_ref[...]
        acc_ref[...] += jnp.sum(xv, axis=0)
        cache_ref[pl.ds(t * tile, tile)] = xv

    @pl.when(t == n_t)
    def _():
        avg = acc_ref[...] * inv_hw
        h = lax.dot_general(avg, w1_ref[...], (((1,), (1,)), ((), ())),
                            preferred_element_type=jnp.float32) + b1_ref[...]
        h = jnp.where(h >= 0.0, h, 0.2 * h)
        s = jnp.dot(h, w2t_ref[...],
                    preferred_element_type=jnp.float32) + b2_ref[...]
        g_ref[...] = jax.nn.sigmoid(s)

    @pl.when(t >= n_t)
    def _():
        xv = cache_ref[pl.ds((t - n_t) * tile, tile)]
        o_ref[...] = (xv * g_ref[...]).astype(o_ref.dtype)


def _pick_hw_tile(hw, n, c, itemsize, budget_bytes):
    best = 1
    for t in range(1, hw + 1):
        if hw % t:
            continue
        if t * n * c * itemsize <= budget_bytes:
            best = t
    return best


@jax.jit
def _se_forward(x_nchw, w1, b1, w2, b2):
    N, C, H, W = x_nchw.shape
    Cr = w1.shape[0]
    HW = H * W
    Nh = N // 2

    xs = jnp.transpose(x_nchw, (2, 3, 0, 1)).reshape(HW, N, C)
    w2t = w2.T
    b1r = b1.reshape(1, Cr)
    b2r = b2.reshape(1, C)

    T = _pick_hw_tile(HW, Nh, C, x_nchw.dtype.itemsize, 3 * _MIB)
    nT = HW // T

    out = pl.pallas_call(
        functools.partial(_se_kernel, n_t=nT, tile=T, inv_hw=1.0 / HW),
        out_shape=jax.ShapeDtypeStruct((HW, N, C), x_nchw.dtype),
        grid=(2, 2 * nT),
        in_specs=[
            pl.BlockSpec((T, Nh, C),
                         lambda i, t: (jnp.minimum(t, nT - 1), i, 0)),
            pl.BlockSpec((Cr, C), lambda i, t: (0, 0)),
            pl.BlockSpec((1, Cr), lambda i, t: (0, 0)),
            pl.BlockSpec((Cr, C), lambda i, t: (0, 0)),
            pl.BlockSpec((1, C), lambda i, t: (0, 0)),
        ],
        out_specs=pl.BlockSpec((T, Nh, C),
                               lambda i, t: (jnp.maximum(t - nT, 0), i, 0)),
        scratch_shapes=[
            pltpu.VMEM((HW, Nh, C), jnp.float32),
            pltpu.VMEM((Nh, C), jnp.float32),
            pltpu.VMEM((Nh, C), jnp.float32),
        ],
        compiler_params=pltpu.CompilerParams(
            dimension_semantics=("parallel", "arbitrary"),
            vmem_limit_bytes=56 * _MIB),
    )(xs, w1, b1r, w2t, b2r)

    return out.reshape(H, W, N, C).transpose(2, 3, 0, 1)


def kernel(x_nchw, w1, b1, w2, b2):
    return _se_forward(x_nchw, w1, b1, w2, b2)


# speedup vs baseline: 6.9254x; 1.1775x over previous
import functools

import jax
import jax.numpy as jnp
from jax import lax
from jax.experimental import pallas as pl
from jax.experimental.pallas import tpu as pltpu

_MIB = 1024 * 1024


def _se_kernel(x_ref, w1_ref, b1_ref, w2t_ref, b2_ref, o_ref,
               cache_ref, acc_ref, g_ref, *, n_t, tile, inv_hw):
    t = pl.program_id(1)

    @pl.when(t == 0)
    def _():
        acc_ref[...] = jnp.zeros_like(acc_ref)

    @pl.when(t < n_t)
    def _():
        xv = x_ref[...]
        acc_ref[...] += jnp.sum(xv, axis=0)
        cache_ref[pl.ds(t * tile, tile)] = xv.astype(cache_ref.dtype)

    @pl.when(t == n_t)
    def _():
        avg = acc_ref[...] * inv_hw
        h = lax.dot_general(avg, w1_ref[...], (((1,), (1,)), ((), ())),
                            preferred_element_type=jnp.float32) + b1_ref[...]
        h = jnp.where(h >= 0.0, h, 0.2 * h)
        s = jnp.dot(h, w2t_ref[...],
                    preferred_element_type=jnp.float32) + b2_ref[...]
        g_ref[...] = jax.nn.sigmoid(s)

    @pl.when(t >= n_t)
    def _():
        xv = cache_ref[pl.ds((t - n_t) * tile, tile)].astype(jnp.float32)
        o_ref[...] = (xv * g_ref[...]).astype(o_ref.dtype)


def _pick_hw_tile(hw, n, c, itemsize, budget_bytes):
    best = 1
    for t in range(1, hw + 1):
        if hw % t:
            continue
        if t * n * c * itemsize <= budget_bytes:
            best = t
    return best


@jax.jit
def _se_forward(x_nchw, w1, b1, w2, b2):
    N, C, H, W = x_nchw.shape
    Cr = w1.shape[0]
    HW = H * W
    Nh = N // 2

    xs = jnp.transpose(x_nchw, (2, 3, 0, 1)).reshape(HW, N, C)
    w2t = w2.T
    b1r = b1.reshape(1, Cr)
    b2r = b2.reshape(1, C)

    T = _pick_hw_tile(HW, Nh, C, x_nchw.dtype.itemsize, 6 * _MIB)
    nT = HW // T

    out = pl.pallas_call(
        functools.partial(_se_kernel, n_t=nT, tile=T, inv_hw=1.0 / HW),
        out_shape=jax.ShapeDtypeStruct((HW, N, C), x_nchw.dtype),
        grid=(2, 2 * nT),
        in_specs=[
            pl.BlockSpec((T, Nh, C),
                         lambda i, t: (jnp.minimum(t, nT - 1), i, 0)),
            pl.BlockSpec((Cr, C), lambda i, t: (0, 0)),
            pl.BlockSpec((1, Cr), lambda i, t: (0, 0)),
            pl.BlockSpec((Cr, C), lambda i, t: (0, 0)),
            pl.BlockSpec((1, C), lambda i, t: (0, 0)),
        ],
        out_specs=pl.BlockSpec((T, Nh, C),
                               lambda i, t: (jnp.maximum(t - nT, 0), i, 0)),
        scratch_shapes=[
            pltpu.VMEM((HW, Nh, C), jnp.bfloat16),
            pltpu.VMEM((Nh, C), jnp.float32),
            pltpu.VMEM((Nh, C), jnp.float32),
        ],
        compiler_params=pltpu.CompilerParams(
            dimension_semantics=("parallel", "arbitrary"),
            vmem_limit_bytes=58 * _MIB),
    )(xs, w1, b1r, w2t, b2r)

    return out.reshape(H, W, N, C).transpose(2, 3, 0, 1)


def kernel(x_nchw, w1, b1, w2, b2):
    return _se_forward(x_nchw, w1, b1, w2, b2)


# speedup vs baseline: 6.9338x; 1.0012x over previous
import functools

import jax
import jax.numpy as jnp
from jax import lax
from jax.experimental import pallas as pl
from jax.experimental.pallas import tpu as pltpu

_MIB = 1024 * 1024


def _se_kernel(x_ref, w1_ref, b1_ref, w2t_ref, b2_ref, o_ref,
               cache_ref, acc_ref, g_ref, *, n_ta, tile_a, tile_b, inv_hw):
    t = pl.program_id(1)

    @pl.when(t == 0)
    def _():
        acc_ref[...] = jnp.zeros_like(acc_ref)

    @pl.when(t < n_ta)
    def _():
        xv = x_ref[...]
        acc_ref[...] += jnp.sum(xv, axis=0)
        cache_ref[pl.ds(t * tile_a, tile_a)] = xv.astype(cache_ref.dtype)

    @pl.when(t == n_ta)
    def _():
        avg = acc_ref[...] * inv_hw
        h = lax.dot_general(avg, w1_ref[...], (((1,), (1,)), ((), ())),
                            preferred_element_type=jnp.float32) + b1_ref[...]
        h = jnp.where(h >= 0.0, h, 0.2 * h)
        s = jnp.dot(h, w2t_ref[...],
                    preferred_element_type=jnp.float32) + b2_ref[...]
        g_ref[...] = jax.nn.sigmoid(s)

    @pl.when(t >= n_ta)
    def _():
        xv = cache_ref[pl.ds((t - n_ta) * tile_b, tile_b)].astype(jnp.float32)
        o_ref[...] = (xv * g_ref[...]).astype(o_ref.dtype)


def _pick_hw_tile(hw, n, c, itemsize, budget_bytes):
    best = 1
    for t in range(1, hw + 1):
        if hw % t:
            continue
        if t * n * c * itemsize <= budget_bytes:
            best = t
    return best


@jax.jit
def _se_forward(x_nchw, w1, b1, w2, b2):
    N, C, H, W = x_nchw.shape
    Cr = w1.shape[0]
    HW = H * W
    Nh = N // 2

    xs = jnp.transpose(x_nchw, (2, 3, 0, 1)).reshape(HW, N, C)
    w2t = w2.T
    b1r = b1.reshape(1, Cr)
    b2r = b2.reshape(1, C)

    Ta = _pick_hw_tile(HW, Nh, C, x_nchw.dtype.itemsize, 10 * _MIB)
    Tb = _pick_hw_tile(HW, Nh, C, x_nchw.dtype.itemsize, 6 * _MIB)
    nTa = HW // Ta
    nTb = HW // Tb

    out = pl.pallas_call(
        functools.partial(_se_kernel, n_ta=nTa, tile_a=Ta, tile_b=Tb,
                          inv_hw=1.0 / HW),
        out_shape=jax.ShapeDtypeStruct((HW, N, C), x_nchw.dtype),
        grid=(2, nTa + nTb),
        in_specs=[
            pl.BlockSpec((Ta, Nh, C),
                         lambda i, t: (jnp.minimum(t, nTa - 1), i, 0)),
            pl.BlockSpec((Cr, C), lambda i, t: (0, 0)),
            pl.BlockSpec((1, Cr), lambda i, t: (0, 0)),
            pl.BlockSpec((Cr, C), lambda i, t: (0, 0)),
            pl.BlockSpec((1, C), lambda i, t: (0, 0)),
        ],
        out_specs=pl.BlockSpec((Tb, Nh, C),
                               lambda i, t: (jnp.maximum(t - nTa, 0), i, 0)),
        scratch_shapes=[
            pltpu.VMEM((HW, Nh, C), jnp.bfloat16),
            pltpu.VMEM((Nh, C), jnp.float32),
            pltpu.VMEM((Nh, C), jnp.float32),
        ],
        compiler_params=pltpu.CompilerParams(
            dimension_semantics=("parallel", "arbitrary"),
            vmem_limit_bytes=58 * _MIB),
    )(xs, w1, b1r, w2t, b2r)

    return out.reshape(H, W, N, C).transpose(2, 3, 0, 1)


def kernel(x_nchw, w1, b1, w2, b2):
    return _se_forward(x_nchw, w1, b1, w2, b2)
